# baseline (speedup 1.0000x reference)
import jax
import jax.numpy as jnp
from jax.experimental import pallas as pl
import jax.experimental.pallas.tpu as pltpu

_TOKENS = 2048
_D_IN = 768
_BASE = 128
_CH = 128
_NBUF = 3
_NCH = _TOKENS // _CH


def _copy(buf, o_ref, sems, c):
    slot = c % _NBUF
    return pltpu.make_async_copy(
        buf.at[slot],
        o_ref.at[pl.ds(c * _CH, _CH), :],
        sems.at[slot],
    )


def _pkm_kernel(x_ref, w_ref, b_ref, o_ref, buf, sems):
    for c in range(_NCH):
        slot = c % _NBUF
        if c >= _NBUF:
            _copy(buf, o_ref, sems, c - _NBUF).wait()
        xs = jax.lax.dot_general(
            x_ref[pl.ds(c * _CH, _CH), :], w_ref[:],
            (((1,), (1,)), ((), ())),
            preferred_element_type=jnp.float32,
        ) + b_ref[:]
        x1 = xs[:, :_BASE]
        x2 = xs[:, _BASE:]
        for i in range(_BASE):
            buf[slot, :, i * _BASE:(i + 1) * _BASE] = x1[:, i:i + 1] + x2
        _copy(buf, o_ref, sems, c).start()
    for c in range(_NCH - _NBUF, _NCH):
        _copy(buf, o_ref, sems, c).wait()


def kernel(x, W, b):
    b2 = b.reshape(1, 2 * _BASE)
    return pl.pallas_call(
        _pkm_kernel,
        in_specs=[
            pl.BlockSpec((_TOKENS, _D_IN), lambda: (0, 0)),
            pl.BlockSpec((2 * _BASE, _D_IN), lambda: (0, 0)),
            pl.BlockSpec((1, 2 * _BASE), lambda: (0, 0)),
        ],
        out_specs=pl.BlockSpec(memory_space=pltpu.MemorySpace.HBM),
        out_shape=jax.ShapeDtypeStruct((_TOKENS, _BASE * _BASE), jnp.float32),
        scratch_shapes=[
            pltpu.VMEM((_NBUF, _CH, _BASE * _BASE), jnp.float32),
            pltpu.SemaphoreType.DMA((_NBUF,)),
        ],
    )(x, W, b2)
